# baseline (speedup 1.0000x reference)
import functools

import jax
import jax.numpy as jnp
from jax import lax
from jax.experimental import pallas as pl
from jax.experimental.pallas import tpu as pltpu
from jax.experimental.pallas import tpu_sc as plsc

_EPS = 1e-5


def _lut_body(atom_ref, pos_ref, count_ref, w_ref, b_ref, out_ref):
    base = atom_ref[...] + pos_ref[...]
    cnt = count_ref[...]
    x = base[:, None, :] + cnt[None, :, :]
    mean = jnp.mean(x, axis=-1, keepdims=True)
    xc = x - mean
    var = jnp.mean(xc * xc, axis=-1, keepdims=True)
    y = xc * lax.rsqrt(var + _EPS)
    out_ref[...] = y * w_ref[...] + b_ref[...]


def _build_lut(atom_table, pos_table, count_padded, ln_weight, ln_bias):
    A, D = atom_table.shape
    CP = count_padded.shape[0]
    return pl.pallas_call(
        _lut_body,
        out_shape=jax.ShapeDtypeStruct((A, CP, D), jnp.float32),
    )(atom_table, pos_table, count_padded,
      ln_weight.reshape(1, D), ln_bias.reshape(1, D))


_CHUNK = 128
_NBUF = 2


def _make_sc_gather(n_rows, B, D, CP, per_w, A):
    info = plsc.get_sparse_core_info()
    NC, L = info.num_cores, info.num_lanes
    n_chunks = per_w // _CHUNK
    n_rounds = n_chunks // _NBUF
    n_vec = per_w // L
    warm_vec = (_NBUF * _CHUNK) // L

    mesh = plsc.VectorSubcoreMesh(core_axis_name="c", subcore_axis_name="s")

    @functools.partial(
        pl.kernel,
        mesh=mesh,
        out_type=[
            jax.ShapeDtypeStruct((n_rows, D), jnp.float32),
            jax.ShapeDtypeStruct((n_rows,), jnp.float32),
        ],
    scratch_types=[
            pltpu.VMEM((per_w,), jnp.int32),
            pltpu.VMEM((per_w,), jnp.float32),
            pltpu.VMEM_SHARED((A * CP, D), jnp.float32),
        ] + [pltpu.VMEM((_CHUNK, D), jnp.float32)] * _NBUF
          + [pltpu.SemaphoreType.DMA] * (2 * _NBUF + 2),
    )
    def sc_gather(fv_hbm, lut_hbm, out_hbm, mask_hbm,
                  idx_v, mask_v, lut_sp, *bufs_sems):
        rows = bufs_sems[:_NBUF]
        gsem = bufs_sems[_NBUF:2 * _NBUF]
        wsem = bufs_sems[2 * _NBUF:3 * _NBUF]
        msem = bufs_sems[3 * _NBUF]
        ssem = bufs_sems[3 * _NBUF + 1]

        sid = lax.axis_index("s")
        wid = sid * NC + lax.axis_index("c")
        base = wid * per_w

        @pl.when(sid == 0)
        def _stage():
            pltpu.make_async_copy(lut_hbm, lut_sp, ssem).start()

        pltpu.sync_copy(fv_hbm.at[pl.ds(base, per_w)], idx_v)

        def vec_body(i, carry):
            v = idx_v[pl.ds(i * L, L)]
            p = base + i * L + lax.iota(jnp.int32, L)
            a = lax.div(p, B)
            c = jnp.minimum(jnp.maximum(v, 0), 200)
            idx_v[pl.ds(i * L, L)] = a * CP + c
            mask_v[pl.ds(i * L, L)] = jnp.where(
                v > 0, jnp.float32(1.0), jnp.float32(0.0))
            return carry

        def fire_gather(j, t):
            pltpu.make_async_copy(
                lut_hbm.at[idx_v.at[pl.ds(j * _CHUNK, _CHUNK)]],
                rows[t], gsem[t]).start()

        def wait_gather(t):
            pltpu.make_async_copy(
                lut_hbm.at[idx_v.at[pl.ds(0, _CHUNK)]],
                rows[t], gsem[t]).wait()

        def fire_write(j, t):
            pltpu.make_async_copy(
                rows[t], out_hbm.at[pl.ds(base + j * _CHUNK, _CHUNK)],
                wsem[t]).start()

        def wait_write(j, t):
            pltpu.make_async_copy(
                rows[t], out_hbm.at[pl.ds(base + j * _CHUNK, _CHUNK)],
                wsem[t]).wait()

        lax.fori_loop(0, warm_vec, vec_body, 0)

        @pl.when(sid == 0)
        def _stage_wait():
            pltpu.make_async_copy(lut_hbm, lut_sp, ssem).wait()

        plsc.subcore_barrier()
        for t in range(_NBUF):
            fire_gather(t, t)
        lax.fori_loop(warm_vec, n_vec, vec_body, 0)
        pltpu.make_async_copy(
            mask_v, mask_hbm.at[pl.ds(base, per_w)], msem).start()

        def round_body(k, carry):
            j0 = k * _NBUF
            for t in range(_NBUF):
                wait_gather(t)
                fire_write(j0 + t, t)
            for t in range(_NBUF):
                wait_write(j0 + t, t)
                fire_gather(j0 + _NBUF + t, t)
            return carry

        lax.fori_loop(0, n_rounds - 1, round_body, 0)

        j0 = (n_rounds - 1) * _NBUF
        for t in range(_NBUF):
            wait_gather(t)
            fire_write(j0 + t, t)
        for t in range(_NBUF):
            wait_write(j0 + t, t)
        pltpu.make_async_copy(
            mask_v, mask_hbm.at[pl.ds(base, per_w)], msem).wait()

    return sc_gather


def kernel(formula_vectors, atom_table, count_table, pos_table,
           ln_weight, ln_bias):
    B, A = formula_vectors.shape
    D = atom_table.shape[1]
    MC1 = count_table.shape[0]
    CP = 208

    count_padded = jnp.zeros((CP, D), jnp.float32).at[:MC1].set(count_table)
    lut = _build_lut(atom_table, pos_table, count_padded, ln_weight, ln_bias)
    lut_flat = lut.reshape(A * CP, D)

    n_rows = B * A
    NW = 32
    per_w = n_rows // NW
    fv_t = formula_vectors.T.reshape(n_rows)

    sc = _make_sc_gather(n_rows, B, D, CP, per_w, A)
    out_flat, mask_flat = sc(fv_t, lut_flat)
    out = out_flat.reshape(A, B, D).transpose(1, 0, 2)
    mask = mask_flat.reshape(A, B).T
    return out, mask


# speedup vs baseline: 1.1136x; 1.1136x over previous
import functools

import jax
import jax.numpy as jnp
from jax import lax
from jax.experimental import pallas as pl
from jax.experimental.pallas import tpu as pltpu
from jax.experimental.pallas import tpu_sc as plsc

_EPS = 1e-5


def _lut_body(atom_ref, pos_ref, count_ref, w_ref, b_ref, out_ref):
    base = atom_ref[...] + pos_ref[...]
    cnt = count_ref[...]
    x = base[:, None, :] + cnt[None, :, :]
    mean = jnp.mean(x, axis=-1, keepdims=True)
    xc = x - mean
    var = jnp.mean(xc * xc, axis=-1, keepdims=True)
    y = xc * lax.rsqrt(var + _EPS)
    out_ref[...] = y * w_ref[...] + b_ref[...]


def _build_lut(atom_table, pos_table, count_padded, ln_weight, ln_bias):
    A, D = atom_table.shape
    CP = count_padded.shape[0]
    return pl.pallas_call(
        _lut_body,
        out_shape=jax.ShapeDtypeStruct((A, CP, D), jnp.float32),
    )(atom_table, pos_table, count_padded,
      ln_weight.reshape(1, D), ln_bias.reshape(1, D))


_CHUNK = 128
_NBUF = 2


def _make_sc_gather(n_rows, B, D, CP, per_w, A):
    info = plsc.get_sparse_core_info()
    NC, L = info.num_cores, info.num_lanes
    n_chunks = per_w // _CHUNK
    n_rounds = n_chunks // _NBUF
    n_vec = per_w // L
    warm_vec = (_NBUF * _CHUNK) // L

    mesh = plsc.VectorSubcoreMesh(core_axis_name="c", subcore_axis_name="s")

    @functools.partial(
        pl.kernel,
        mesh=mesh,
        out_type=[
            jax.ShapeDtypeStruct((n_rows, D), jnp.float32),
            jax.ShapeDtypeStruct((n_rows,), jnp.float32),
        ],
    scratch_types=[
            pltpu.VMEM((per_w,), jnp.int32),
            pltpu.VMEM((per_w,), jnp.float32),
            pltpu.VMEM_SHARED((A * CP, D), jnp.float32),
        ] + [pltpu.VMEM((_CHUNK, D), jnp.float32)] * _NBUF
          + [pltpu.SemaphoreType.DMA] * (2 * _NBUF + 2),
    )
    def sc_gather(fv_hbm, lut_hbm, out_hbm, mask_hbm,
                  idx_v, mask_v, lut_sp, *bufs_sems):
        rows = bufs_sems[:_NBUF]
        gsem = bufs_sems[_NBUF:2 * _NBUF]
        wsem = bufs_sems[2 * _NBUF:3 * _NBUF]
        msem = bufs_sems[3 * _NBUF]
        ssem = bufs_sems[3 * _NBUF + 1]

        sid = lax.axis_index("s")
        wid = sid * NC + lax.axis_index("c")
        base = wid * per_w

        @pl.when(sid == 0)
        def _stage():
            pltpu.make_async_copy(lut_hbm, lut_sp, ssem).start()

        pltpu.sync_copy(fv_hbm.at[pl.ds(base, per_w)], idx_v)

        def vec_body(i, carry):
            v = idx_v[pl.ds(i * L, L)]
            p = base + i * L + lax.iota(jnp.int32, L)
            a = lax.div(p, B)
            c = jnp.minimum(jnp.maximum(v, 0), 200)
            idx_v[pl.ds(i * L, L)] = a * CP + c
            mask_v[pl.ds(i * L, L)] = jnp.where(
                v > 0, jnp.float32(1.0), jnp.float32(0.0))
            return carry

        def _lut(t):
            return lut_hbm if t == 0 else lut_sp

        def fire_gather(j, t):
            pltpu.make_async_copy(
                _lut(t).at[idx_v.at[pl.ds(j * _CHUNK, _CHUNK)]],
                rows[t], gsem[t]).start()

        def wait_gather(t):
            pltpu.make_async_copy(
                _lut(t).at[idx_v.at[pl.ds(0, _CHUNK)]],
                rows[t], gsem[t]).wait()

        def fire_write(j, t):
            pltpu.make_async_copy(
                rows[t], out_hbm.at[pl.ds(base + j * _CHUNK, _CHUNK)],
                wsem[t]).start()

        def wait_write(j, t):
            pltpu.make_async_copy(
                rows[t], out_hbm.at[pl.ds(base + j * _CHUNK, _CHUNK)],
                wsem[t]).wait()

        lax.fori_loop(0, warm_vec, vec_body, 0)

        @pl.when(sid == 0)
        def _stage_wait():
            pltpu.make_async_copy(lut_hbm, lut_sp, ssem).wait()

        plsc.subcore_barrier()
        for t in range(_NBUF):
            fire_gather(t, t)
        lax.fori_loop(warm_vec, n_vec, vec_body, 0)
        pltpu.make_async_copy(
            mask_v, mask_hbm.at[pl.ds(base, per_w)], msem).start()

        def round_body(k, carry):
            j0 = k * _NBUF
            for t in range(_NBUF):
                wait_gather(t)
                fire_write(j0 + t, t)
            for t in range(_NBUF):
                wait_write(j0 + t, t)
                fire_gather(j0 + _NBUF + t, t)
            return carry

        lax.fori_loop(0, n_rounds - 1, round_body, 0)

        j0 = (n_rounds - 1) * _NBUF
        for t in range(_NBUF):
            wait_gather(t)
            fire_write(j0 + t, t)
        for t in range(_NBUF):
            wait_write(j0 + t, t)
        pltpu.make_async_copy(
            mask_v, mask_hbm.at[pl.ds(base, per_w)], msem).wait()

    return sc_gather


def kernel(formula_vectors, atom_table, count_table, pos_table,
           ln_weight, ln_bias):
    B, A = formula_vectors.shape
    D = atom_table.shape[1]
    MC1 = count_table.shape[0]
    CP = 208

    count_padded = jnp.zeros((CP, D), jnp.float32).at[:MC1].set(count_table)
    lut = _build_lut(atom_table, pos_table, count_padded, ln_weight, ln_bias)
    lut_flat = lut.reshape(A * CP, D)

    n_rows = B * A
    NW = 32
    per_w = n_rows // NW
    fv_t = formula_vectors.T.reshape(n_rows)

    sc = _make_sc_gather(n_rows, B, D, CP, per_w, A)
    out_flat, mask_flat = sc(fv_t, lut_flat)
    out = out_flat.reshape(A, B, D).transpose(1, 0, 2)
    mask = mask_flat.reshape(A, B).T
    return out, mask


# speedup vs baseline: 1.2805x; 1.1498x over previous
import functools

import jax
import jax.numpy as jnp
from jax import lax
from jax.experimental import pallas as pl
from jax.experimental.pallas import tpu as pltpu
from jax.experimental.pallas import tpu_sc as plsc

_EPS = 1e-5


def _lut_body(atom_ref, pos_ref, count_ref, w_ref, b_ref, out_ref):
    base = atom_ref[...] + pos_ref[...]
    cnt = count_ref[...]
    x = base[:, None, :] + cnt[None, :, :]
    mean = jnp.mean(x, axis=-1, keepdims=True)
    xc = x - mean
    var = jnp.mean(xc * xc, axis=-1, keepdims=True)
    y = xc * lax.rsqrt(var + _EPS)
    out_ref[...] = y * w_ref[...] + b_ref[...]


def _build_lut(atom_table, pos_table, count_padded, ln_weight, ln_bias):
    A, D = atom_table.shape
    CP = count_padded.shape[0]
    return pl.pallas_call(
        _lut_body,
        out_shape=jax.ShapeDtypeStruct((A, CP, D), jnp.float32),
    )(atom_table, pos_table, count_padded,
      ln_weight.reshape(1, D), ln_bias.reshape(1, D))


_CHUNK = 128
_NBUF = 2


def _make_sc_gather(n_rows, B, D, CP, per_w, A):
    info = plsc.get_sparse_core_info()
    NC, L = info.num_cores, info.num_lanes
    n_chunks = per_w // _CHUNK
    n_rounds = n_chunks // _NBUF
    n_vec = per_w // L
    warm_vec = (_NBUF * _CHUNK) // L

    mesh = plsc.VectorSubcoreMesh(core_axis_name="c", subcore_axis_name="s")

    @functools.partial(
        pl.kernel,
        mesh=mesh,
        out_type=[
            jax.ShapeDtypeStruct((n_rows, D), jnp.float32),
            jax.ShapeDtypeStruct((n_rows,), jnp.float32),
        ],
    scratch_types=[
            pltpu.VMEM((per_w,), jnp.int32),
            pltpu.VMEM((per_w,), jnp.float32),
            pltpu.VMEM_SHARED((A * CP, D), jnp.float32),
        ] + [pltpu.VMEM((_CHUNK, D), jnp.float32)] * _NBUF
          + [pltpu.SemaphoreType.DMA] * (2 * _NBUF + 2),
    )
    def sc_gather(fv_hbm, lut_hbm, out_hbm, mask_hbm,
                  idx_v, mask_v, lut_sp, *bufs_sems):
        rows = bufs_sems[:_NBUF]
        gsem = bufs_sems[_NBUF:2 * _NBUF]
        wsem = bufs_sems[2 * _NBUF:3 * _NBUF]
        msem = bufs_sems[3 * _NBUF]
        ssem = bufs_sems[3 * _NBUF + 1]

        sid = lax.axis_index("s")
        wid = sid * NC + lax.axis_index("c")
        base = wid * per_w

        @pl.when(sid == 0)
        def _stage():
            pltpu.make_async_copy(lut_hbm, lut_sp, ssem).start()

        pltpu.sync_copy(fv_hbm.at[pl.ds(base, per_w)], idx_v)

        def vec_body(i, carry):
            v = idx_v[pl.ds(i * L, L)]
            p = base + i * L + lax.iota(jnp.int32, L)
            a = lax.div(p, B)
            c = jnp.minimum(jnp.maximum(v, 0), 200)
            idx_v[pl.ds(i * L, L)] = a * CP + c
            mask_v[pl.ds(i * L, L)] = jnp.where(
                v > 0, jnp.float32(1.0), jnp.float32(0.0))
            return carry

        def fire_gather(j, t):
            pltpu.make_async_copy(
                lut_sp.at[idx_v.at[pl.ds(j * _CHUNK, _CHUNK)]],
                rows[t], gsem[t]).start()

        def wait_gather(t):
            pltpu.make_async_copy(
                lut_sp.at[idx_v.at[pl.ds(0, _CHUNK)]],
                rows[t], gsem[t]).wait()

        def fire_write(j, t):
            pltpu.make_async_copy(
                rows[t], out_hbm.at[pl.ds(base + j * _CHUNK, _CHUNK)],
                wsem[t]).start()

        def wait_write(j, t):
            pltpu.make_async_copy(
                rows[t], out_hbm.at[pl.ds(base + j * _CHUNK, _CHUNK)],
                wsem[t]).wait()

        lax.fori_loop(0, warm_vec, vec_body, 0)

        @pl.when(sid == 0)
        def _stage_wait():
            pltpu.make_async_copy(lut_hbm, lut_sp, ssem).wait()

        plsc.subcore_barrier()
        for t in range(_NBUF):
            fire_gather(t, t)
        lax.fori_loop(warm_vec, n_vec, vec_body, 0)
        pltpu.make_async_copy(
            mask_v, mask_hbm.at[pl.ds(base, per_w)], msem).start()

        def round_body(k, carry):
            j0 = k * _NBUF
            for t in range(_NBUF):
                wait_gather(t)
                fire_write(j0 + t, t)
            for t in range(_NBUF):
                wait_write(j0 + t, t)
                fire_gather(j0 + _NBUF + t, t)
            return carry

        lax.fori_loop(0, n_rounds - 1, round_body, 0)

        j0 = (n_rounds - 1) * _NBUF
        for t in range(_NBUF):
            wait_gather(t)
            fire_write(j0 + t, t)
        for t in range(_NBUF):
            wait_write(j0 + t, t)
        pltpu.make_async_copy(
            mask_v, mask_hbm.at[pl.ds(base, per_w)], msem).wait()

    return sc_gather


def kernel(formula_vectors, atom_table, count_table, pos_table,
           ln_weight, ln_bias):
    B, A = formula_vectors.shape
    D = atom_table.shape[1]
    MC1 = count_table.shape[0]
    CP = 208

    count_padded = jnp.zeros((CP, D), jnp.float32).at[:MC1].set(count_table)
    lut = _build_lut(atom_table, pos_table, count_padded, ln_weight, ln_bias)
    lut_flat = lut.reshape(A * CP, D)

    n_rows = B * A
    NW = 32
    per_w = n_rows // NW
    fv_t = formula_vectors.T.reshape(n_rows)

    sc = _make_sc_gather(n_rows, B, D, CP, per_w, A)
    out_flat, mask_flat = sc(fv_t, lut_flat)
    out = out_flat.reshape(A, B, D).transpose(1, 0, 2)
    mask = mask_flat.reshape(A, B).T
    return out, mask
